# baseline (speedup 1.0000x reference)
import functools

import jax
import jax.numpy as jnp
from jax import lax
from jax.experimental import pallas as pl
from jax.experimental.pallas import tpu as pltpu
from jax.experimental.pallas import tpu_sc as plsc

N = 10000
E = 320000
F_IN = 128

NC = 2
NS = 16
NW = NC * NS

EPW = E // NW
CHUNK = 80
NCHUNK = EPW // CHUNK
NPAIR = (NCHUNK - 1) // 2
RPS = 624
RPS_TAIL = N - NS * RPS

DEG_PAD = 10240
DEG_W = 8
RPSD = DEG_PAD // NS

_mesh = plsc.VectorSubcoreMesh(
    core_axis_name="c", subcore_axis_name="s", num_cores=NC, num_subcores=NS)

_sc_params = pltpu.CompilerParams(use_tc_tiling_on_sc=False)


@functools.partial(
    pl.kernel,
    out_type=[jax.ShapeDtypeStruct((DEG_PAD, DEG_W), jnp.float32),
              jax.ShapeDtypeStruct((DEG_PAD, DEG_W), jnp.float32)],
    mesh=_mesh,
    scratch_types=[
        pltpu.VMEM((NCHUNK, CHUNK), jnp.int32),
        pltpu.VMEM((CHUNK, DEG_W), jnp.float32),
        pltpu.VMEM_SHARED((DEG_PAD, DEG_W), jnp.float32),
    ],
    compiler_params=_sc_params,
)
def _deg_kernel(dst_hbm, ones_hbm, deg0_out, deg1_out, idx_v, ones_v, acc_sh):
    cid = lax.axis_index("c")
    sid = lax.axis_index("s")
    wid = sid * NC + cid
    pltpu.sync_copy(dst_hbm.at[wid], idx_v)
    pltpu.sync_copy(ones_hbm.at[pl.ds(0, CHUNK)], ones_v)
    pltpu.sync_copy(ones_hbm.at[pl.ds(sid * RPSD, RPSD)],
                    acc_sh.at[pl.ds(sid * RPSD, RPSD)])
    plsc.subcore_barrier()

    def body(g, carry):
        pltpu.sync_copy(ones_v, acc_sh.at[idx_v.at[g]], add=True)
        return carry

    lax.fori_loop(0, NCHUNK, body, 0)
    plsc.subcore_barrier()

    @pl.when(cid == 0)
    def _():
        pltpu.sync_copy(acc_sh.at[pl.ds(sid * RPSD, RPSD)],
                        deg0_out.at[pl.ds(sid * RPSD, RPSD)])

    @pl.when(cid == 1)
    def _():
        pltpu.sync_copy(acc_sh.at[pl.ds(sid * RPSD, RPSD)],
                        deg1_out.at[pl.ds(sid * RPSD, RPSD)])


def _make_edge_kernel(d_feat):
    @functools.partial(
        pl.kernel,
        out_type=[jax.ShapeDtypeStruct((N, d_feat), jnp.float32),
                  jax.ShapeDtypeStruct((N, d_feat), jnp.float32)],
        mesh=_mesh,
        scratch_types=[
            pltpu.VMEM((NCHUNK, CHUNK), jnp.int32),
            pltpu.VMEM((NCHUNK, CHUNK), jnp.int32),
            pltpu.VMEM((CHUNK, d_feat), jnp.float32),
            pltpu.VMEM((CHUNK, d_feat), jnp.float32),
            pltpu.VMEM_SHARED((N, d_feat), jnp.float32),
            pltpu.SemaphoreType.DMA,
            pltpu.SemaphoreType.DMA,
        ],
        compiler_params=_sc_params,
    )
    def edge_kernel(hs_hbm, src_hbm, dst_hbm, acc0_out, acc1_out,
                    src_v, dst_v, rows0, rows1, acc_sh, sem0, sem1):
        cid = lax.axis_index("c")
        sid = lax.axis_index("s")
        wid = sid * NC + cid
        pltpu.sync_copy(src_hbm.at[wid], src_v)
        pltpu.sync_copy(dst_hbm.at[wid], dst_v)
        pltpu.sync_copy(hs_hbm.at[pl.ds(sid * RPS, RPS)],
                        acc_sh.at[pl.ds(sid * RPS, RPS)])

        @pl.when(sid == NS - 1)
        def _():
            pltpu.sync_copy(hs_hbm.at[pl.ds(NS * RPS, RPS_TAIL)],
                            acc_sh.at[pl.ds(NS * RPS, RPS_TAIL)])

        plsc.subcore_barrier()

        pltpu.async_copy(hs_hbm.at[src_v.at[0]], rows0, sem0)

        def body(p, carry):
            g0 = 2 * p
            g1 = g0 + 1
            pltpu.async_copy(hs_hbm.at[src_v.at[g1]], rows1, sem1)
            pltpu.make_async_copy(hs_hbm.at[src_v.at[g0]], rows0, sem0).wait()
            pltpu.sync_copy(rows0, acc_sh.at[dst_v.at[g0]], add=True)
            pltpu.async_copy(hs_hbm.at[src_v.at[g0 + 2]], rows0, sem0)
            pltpu.make_async_copy(hs_hbm.at[src_v.at[g1]], rows1, sem1).wait()
            pltpu.sync_copy(rows1, acc_sh.at[dst_v.at[g1]], add=True)
            return carry

        lax.fori_loop(0, NPAIR, body, 0)
        g_last = NCHUNK - 1
        pltpu.make_async_copy(hs_hbm.at[src_v.at[g_last]], rows0, sem0).wait()
        pltpu.sync_copy(rows0, acc_sh.at[dst_v.at[g_last]], add=True)

        plsc.subcore_barrier()

        @pl.when(cid == 0)
        def _():
            pltpu.sync_copy(acc_sh.at[pl.ds(sid * RPS, RPS)],
                            acc0_out.at[pl.ds(sid * RPS, RPS)])

            @pl.when(sid == NS - 1)
            def _():
                pltpu.sync_copy(acc_sh.at[pl.ds(NS * RPS, RPS_TAIL)],
                                acc0_out.at[pl.ds(NS * RPS, RPS_TAIL)])

        @pl.when(cid == 1)
        def _():
            pltpu.sync_copy(acc_sh.at[pl.ds(sid * RPS, RPS)],
                            acc1_out.at[pl.ds(sid * RPS, RPS)])

            @pl.when(sid == NS - 1)
            def _():
                pltpu.sync_copy(acc_sh.at[pl.ds(NS * RPS, RPS_TAIL)],
                                acc1_out.at[pl.ds(NS * RPS, RPS_TAIL)])

    return edge_kernel


_edge64 = _make_edge_kernel(64)
_edge32 = _make_edge_kernel(32)


BN = 2000


def _dinv(deg0_ref, deg1_ref):
    return lax.rsqrt(deg0_ref[:, :1] + deg1_ref[:, :1] - 1.0)


def _deg_spec():
    return pl.BlockSpec((BN, DEG_W), lambda i: (i, 0))


def _tc_first(deg0, deg1, x, w0):
    def body(deg0_ref, deg1_ref, x_ref, w_ref, out_ref):
        dinv = _dinv(deg0_ref, deg1_ref)
        h = jnp.dot(x_ref[...], w_ref[...], preferred_element_type=jnp.float32)
        out_ref[...] = dinv * h

    d_out = w0.shape[1]
    return pl.pallas_call(
        body,
        grid=(N // BN,),
        in_specs=[
            _deg_spec(),
            _deg_spec(),
            pl.BlockSpec((BN, F_IN), lambda i: (i, 0)),
            pl.BlockSpec((F_IN, d_out), lambda i: (0, 0)),
        ],
        out_specs=pl.BlockSpec((BN, d_out), lambda i: (i, 0)),
        out_shape=jax.ShapeDtypeStruct((N, d_out), jnp.float32),
    )(deg0, deg1, x, w0)


def _tc_mid(deg0, deg1, acc0, acc1, hs, w, b):
    d_in = hs.shape[1]
    d_out = w.shape[1]

    def body(deg0_ref, deg1_ref, a0_ref, a1_ref, hs_ref, w_ref, b_ref,
             out_ref):
        dinv = _dinv(deg0_ref, deg1_ref)
        a = a0_ref[...] + a1_ref[...] - hs_ref[...]
        h = jnp.maximum(dinv * a + b_ref[...], 0.0)
        hn = jnp.dot(h, w_ref[...], preferred_element_type=jnp.float32)
        out_ref[...] = dinv * hn

    return pl.pallas_call(
        body,
        grid=(N // BN,),
        in_specs=[
            _deg_spec(),
            _deg_spec(),
            pl.BlockSpec((BN, d_in), lambda i: (i, 0)),
            pl.BlockSpec((BN, d_in), lambda i: (i, 0)),
            pl.BlockSpec((BN, d_in), lambda i: (i, 0)),
            pl.BlockSpec((d_in, d_out), lambda i: (0, 0)),
            pl.BlockSpec((1, d_in), lambda i: (0, 0)),
        ],
        out_specs=pl.BlockSpec((BN, d_out), lambda i: (i, 0)),
        out_shape=jax.ShapeDtypeStruct((N, d_out), jnp.float32),
    )(deg0, deg1, acc0, acc1, hs, w, b)


def _tc_last(deg0, deg1, acc0, acc1, hs, b):
    d_in = hs.shape[1]

    def body(deg0_ref, deg1_ref, a0_ref, a1_ref, hs_ref, b_ref, out_ref):
        dinv = _dinv(deg0_ref, deg1_ref)
        a = a0_ref[...] + a1_ref[...] - hs_ref[...]
        out_ref[...] = jnp.maximum(dinv * a + b_ref[...], 0.0)

    return pl.pallas_call(
        body,
        grid=(N // BN,),
        in_specs=[
            _deg_spec(),
            _deg_spec(),
            pl.BlockSpec((BN, d_in), lambda i: (i, 0)),
            pl.BlockSpec((BN, d_in), lambda i: (i, 0)),
            pl.BlockSpec((BN, d_in), lambda i: (i, 0)),
            pl.BlockSpec((1, d_in), lambda i: (0, 0)),
        ],
        out_specs=pl.BlockSpec((BN, d_in), lambda i: (i, 0)),
        out_shape=jax.ShapeDtypeStruct((N, d_in), jnp.float32),
    )(deg0, deg1, acc0, acc1, hs, b)


def kernel(x, edge_index, batch, W0, b0, W1, b1, W2, b2):
    src = edge_index[0].reshape(NW, NCHUNK, CHUNK)
    dst = edge_index[1].reshape(NW, NCHUNK, CHUNK)
    ones = jnp.ones((DEG_PAD, DEG_W), jnp.float32)

    deg0, deg1 = _deg_kernel(dst, ones)

    hs1 = _tc_first(deg0, deg1, x, W0)
    a0, a1 = _edge64(hs1, src, dst)
    hs2 = _tc_mid(deg0, deg1, a0, a1, hs1, W1, b0.reshape(1, -1))
    a0, a1 = _edge32(hs2, src, dst)
    hs3 = _tc_mid(deg0, deg1, a0, a1, hs2, W2, b1.reshape(1, -1))
    a0, a1 = _edge32(hs3, src, dst)
    return _tc_last(deg0, deg1, a0, a1, hs3, b2.reshape(1, -1))


# speedup vs baseline: 1.1350x; 1.1350x over previous
import functools

import jax
import jax.numpy as jnp
from jax import lax
from jax.experimental import pallas as pl
from jax.experimental.pallas import tpu as pltpu
from jax.experimental.pallas import tpu_sc as plsc

N = 10000
E = 320000
F_IN = 128

NC = 2
NS = 16
NW = NC * NS

EPW = E // NW
CHUNK = 128
NCHUNK = 80
NPAIR = NCHUNK // 2
EPW_PAD = NCHUNK * CHUNK
PADW = EPW_PAD - EPW
ACC_N = N + PADW
RPS = 624
RPS_TAIL = N - NS * RPS

DEG_PAD = 10240
DEG_W = 8
RPSD = DEG_PAD // NS

_mesh = plsc.VectorSubcoreMesh(
    core_axis_name="c", subcore_axis_name="s", num_cores=NC, num_subcores=NS)

_sc_params = pltpu.CompilerParams(use_tc_tiling_on_sc=False)


def _fill_pad(idx_v, base_val):
    for i in range(PADW // 16):
        idx_v[pl.ds(EPW + 16 * i, 16)] = (
            lax.iota(jnp.int32, 16) + (base_val + 16 * i))


@functools.partial(
    pl.kernel,
    out_type=[jax.ShapeDtypeStruct((DEG_PAD, DEG_W), jnp.float32),
              jax.ShapeDtypeStruct((DEG_PAD, DEG_W), jnp.float32)],
    mesh=_mesh,
    scratch_types=[
        pltpu.VMEM((EPW_PAD,), jnp.int32),
        pltpu.VMEM((CHUNK, DEG_W), jnp.float32),
        pltpu.VMEM_SHARED((DEG_PAD, DEG_W), jnp.float32),
    ],
    compiler_params=_sc_params,
)
def _deg_kernel(dst_hbm, ones_hbm, deg0_out, deg1_out, idx_v, ones_v, acc_sh):
    cid = lax.axis_index("c")
    sid = lax.axis_index("s")
    wid = sid * NC + cid
    pltpu.sync_copy(dst_hbm.at[wid], idx_v.at[pl.ds(0, EPW)])
    _fill_pad(idx_v, N)
    pltpu.sync_copy(ones_hbm.at[pl.ds(0, CHUNK)], ones_v)
    pltpu.sync_copy(ones_hbm.at[pl.ds(sid * RPSD, RPSD)],
                    acc_sh.at[pl.ds(sid * RPSD, RPSD)])
    plsc.subcore_barrier()

    def body(g, carry):
        pltpu.sync_copy(ones_v,
                        acc_sh.at[idx_v.at[pl.ds(g * CHUNK, CHUNK)]],
                        add=True)
        return carry

    lax.fori_loop(0, NCHUNK, body, 0)
    plsc.subcore_barrier()

    @pl.when(cid == 0)
    def _():
        pltpu.sync_copy(acc_sh.at[pl.ds(sid * RPSD, RPSD)],
                        deg0_out.at[pl.ds(sid * RPSD, RPSD)])

    @pl.when(cid == 1)
    def _():
        pltpu.sync_copy(acc_sh.at[pl.ds(sid * RPSD, RPSD)],
                        deg1_out.at[pl.ds(sid * RPSD, RPSD)])


def _make_edge_kernel(d_feat):
    @functools.partial(
        pl.kernel,
        out_type=[jax.ShapeDtypeStruct((N, d_feat), jnp.float32),
                  jax.ShapeDtypeStruct((N, d_feat), jnp.float32)],
        mesh=_mesh,
        scratch_types=[
            pltpu.VMEM((EPW_PAD,), jnp.int32),
            pltpu.VMEM((EPW_PAD,), jnp.int32),
            pltpu.VMEM((CHUNK, d_feat), jnp.float32),
            pltpu.VMEM((CHUNK, d_feat), jnp.float32),
            pltpu.VMEM_SHARED((ACC_N, d_feat), jnp.float32),
            pltpu.SemaphoreType.DMA,
            pltpu.SemaphoreType.DMA,
        ],
        compiler_params=_sc_params,
    )
    def edge_kernel(hs_hbm, src_hbm, dst_hbm, acc0_out, acc1_out,
                    src_v, dst_v, rows0, rows1, acc_sh, sem0, sem1):
        cid = lax.axis_index("c")
        sid = lax.axis_index("s")
        wid = sid * NC + cid
        pltpu.sync_copy(src_hbm.at[wid], src_v.at[pl.ds(0, EPW)])
        pltpu.sync_copy(dst_hbm.at[wid], dst_v.at[pl.ds(0, EPW)])
        _fill_pad(src_v, 0)
        _fill_pad(dst_v, N)
        pltpu.sync_copy(hs_hbm.at[pl.ds(sid * RPS, RPS)],
                        acc_sh.at[pl.ds(sid * RPS, RPS)])

        @pl.when(sid == NS - 1)
        def _():
            pltpu.sync_copy(hs_hbm.at[pl.ds(NS * RPS, RPS_TAIL)],
                            acc_sh.at[pl.ds(NS * RPS, RPS_TAIL)])

        plsc.subcore_barrier()

        def s_idx(g):
            return src_v.at[pl.ds(g * CHUNK, CHUNK)]

        def d_idx(g):
            return dst_v.at[pl.ds(g * CHUNK, CHUNK)]

        pltpu.async_copy(hs_hbm.at[s_idx(0)], rows0, sem0)

        def body(p, carry):
            g0 = 2 * p
            g1 = g0 + 1
            pltpu.async_copy(hs_hbm.at[s_idx(g1)], rows1, sem1)
            pltpu.make_async_copy(hs_hbm.at[s_idx(g0)], rows0, sem0).wait()
            pltpu.sync_copy(rows0, acc_sh.at[d_idx(g0)], add=True)
            pltpu.async_copy(hs_hbm.at[s_idx(g0 + 2)], rows0, sem0)
            pltpu.make_async_copy(hs_hbm.at[s_idx(g1)], rows1, sem1).wait()
            pltpu.sync_copy(rows1, acc_sh.at[d_idx(g1)], add=True)
            return carry

        lax.fori_loop(0, NPAIR - 1, body, 0)
        ga = NCHUNK - 2
        gb = NCHUNK - 1
        pltpu.async_copy(hs_hbm.at[s_idx(gb)], rows1, sem1)
        pltpu.make_async_copy(hs_hbm.at[s_idx(ga)], rows0, sem0).wait()
        pltpu.sync_copy(rows0, acc_sh.at[d_idx(ga)], add=True)
        pltpu.make_async_copy(hs_hbm.at[s_idx(gb)], rows1, sem1).wait()
        pltpu.sync_copy(rows1, acc_sh.at[d_idx(gb)], add=True)

        plsc.subcore_barrier()

        @pl.when(cid == 0)
        def _():
            pltpu.sync_copy(acc_sh.at[pl.ds(sid * RPS, RPS)],
                            acc0_out.at[pl.ds(sid * RPS, RPS)])

            @pl.when(sid == NS - 1)
            def _():
                pltpu.sync_copy(acc_sh.at[pl.ds(NS * RPS, RPS_TAIL)],
                                acc0_out.at[pl.ds(NS * RPS, RPS_TAIL)])

        @pl.when(cid == 1)
        def _():
            pltpu.sync_copy(acc_sh.at[pl.ds(sid * RPS, RPS)],
                            acc1_out.at[pl.ds(sid * RPS, RPS)])

            @pl.when(sid == NS - 1)
            def _():
                pltpu.sync_copy(acc_sh.at[pl.ds(NS * RPS, RPS_TAIL)],
                                acc1_out.at[pl.ds(NS * RPS, RPS_TAIL)])

    return edge_kernel


_edge64 = _make_edge_kernel(64)
_edge32 = _make_edge_kernel(32)


BN = 2000


def _dinv(deg0_ref, deg1_ref):
    return lax.rsqrt(deg0_ref[:, :1] + deg1_ref[:, :1] - 1.0)


def _deg_spec():
    return pl.BlockSpec((BN, DEG_W), lambda i: (i, 0))


def _tc_first(deg0, deg1, x, w0):
    def body(deg0_ref, deg1_ref, x_ref, w_ref, out_ref):
        dinv = _dinv(deg0_ref, deg1_ref)
        h = jnp.dot(x_ref[...], w_ref[...], preferred_element_type=jnp.float32)
        out_ref[...] = dinv * h

    d_out = w0.shape[1]
    return pl.pallas_call(
        body,
        grid=(N // BN,),
        in_specs=[
            _deg_spec(),
            _deg_spec(),
            pl.BlockSpec((BN, F_IN), lambda i: (i, 0)),
            pl.BlockSpec((F_IN, d_out), lambda i: (0, 0)),
        ],
        out_specs=pl.BlockSpec((BN, d_out), lambda i: (i, 0)),
        out_shape=jax.ShapeDtypeStruct((N, d_out), jnp.float32),
    )(deg0, deg1, x, w0)


def _tc_mid(deg0, deg1, acc0, acc1, hs, w, b):
    d_in = hs.shape[1]
    d_out = w.shape[1]

    def body(deg0_ref, deg1_ref, a0_ref, a1_ref, hs_ref, w_ref, b_ref,
             out_ref):
        dinv = _dinv(deg0_ref, deg1_ref)
        a = a0_ref[...] + a1_ref[...] - hs_ref[...]
        h = jnp.maximum(dinv * a + b_ref[...], 0.0)
        hn = jnp.dot(h, w_ref[...], preferred_element_type=jnp.float32)
        out_ref[...] = dinv * hn

    return pl.pallas_call(
        body,
        grid=(N // BN,),
        in_specs=[
            _deg_spec(),
            _deg_spec(),
            pl.BlockSpec((BN, d_in), lambda i: (i, 0)),
            pl.BlockSpec((BN, d_in), lambda i: (i, 0)),
            pl.BlockSpec((BN, d_in), lambda i: (i, 0)),
            pl.BlockSpec((d_in, d_out), lambda i: (0, 0)),
            pl.BlockSpec((1, d_in), lambda i: (0, 0)),
        ],
        out_specs=pl.BlockSpec((BN, d_out), lambda i: (i, 0)),
        out_shape=jax.ShapeDtypeStruct((N, d_out), jnp.float32),
    )(deg0, deg1, acc0, acc1, hs, w, b)


def _tc_last(deg0, deg1, acc0, acc1, hs, b):
    d_in = hs.shape[1]

    def body(deg0_ref, deg1_ref, a0_ref, a1_ref, hs_ref, b_ref, out_ref):
        dinv = _dinv(deg0_ref, deg1_ref)
        a = a0_ref[...] + a1_ref[...] - hs_ref[...]
        out_ref[...] = jnp.maximum(dinv * a + b_ref[...], 0.0)

    return pl.pallas_call(
        body,
        grid=(N // BN,),
        in_specs=[
            _deg_spec(),
            _deg_spec(),
            pl.BlockSpec((BN, d_in), lambda i: (i, 0)),
            pl.BlockSpec((BN, d_in), lambda i: (i, 0)),
            pl.BlockSpec((BN, d_in), lambda i: (i, 0)),
            pl.BlockSpec((1, d_in), lambda i: (0, 0)),
        ],
        out_specs=pl.BlockSpec((BN, d_in), lambda i: (i, 0)),
        out_shape=jax.ShapeDtypeStruct((N, d_in), jnp.float32),
    )(deg0, deg1, acc0, acc1, hs, b)


def kernel(x, edge_index, batch, W0, b0, W1, b1, W2, b2):
    src = edge_index[0].reshape(NW, EPW)
    dst = edge_index[1].reshape(NW, EPW)
    ones = jnp.ones((DEG_PAD, DEG_W), jnp.float32)

    deg0, deg1 = _deg_kernel(dst, ones)

    hs1 = _tc_first(deg0, deg1, x, W0)
    a0, a1 = _edge64(hs1, src, dst)
    hs2 = _tc_mid(deg0, deg1, a0, a1, hs1, W1, b0.reshape(1, -1))
    a0, a1 = _edge32(hs2, src, dst)
    hs3 = _tc_mid(deg0, deg1, a0, a1, hs2, W2, b1.reshape(1, -1))
    a0, a1 = _edge32(hs3, src, dst)
    return _tc_last(deg0, deg1, a0, a1, hs3, b2.reshape(1, -1))


# speedup vs baseline: 1.3372x; 1.1782x over previous
import functools

import jax
import jax.numpy as jnp
from jax import lax
from jax.experimental import pallas as pl
from jax.experimental.pallas import tpu as pltpu
from jax.experimental.pallas import tpu_sc as plsc

N = 10000
E = 320000
F_IN = 128

NC = 2
NS = 16
NW = NC * NS

EPW = E // NW
CHUNK = 128
NCHUNK = 80
NPAIR = NCHUNK // 2
EPW_PAD = NCHUNK * CHUNK
PADW = EPW_PAD - EPW
ACC_N = N + PADW
RPS = 624
RPS_TAIL = N - NS * RPS

DEG_PAD = 10240
DEG_W = 8
RPSD = DEG_PAD // NS

_mesh = plsc.VectorSubcoreMesh(
    core_axis_name="c", subcore_axis_name="s", num_cores=NC, num_subcores=NS)

_sc_params = pltpu.CompilerParams(use_tc_tiling_on_sc=False)


def _fill_pad(idx_v, base_val):
    for i in range(PADW // 16):
        idx_v[pl.ds(EPW + 16 * i, 16)] = (
            lax.iota(jnp.int32, 16) + (base_val + 16 * i))


@functools.partial(
    pl.kernel,
    out_type=[jax.ShapeDtypeStruct((DEG_PAD, DEG_W), jnp.float32),
              jax.ShapeDtypeStruct((DEG_PAD, DEG_W), jnp.float32)],
    mesh=_mesh,
    scratch_types=[
        pltpu.VMEM((EPW_PAD,), jnp.int32),
        pltpu.VMEM((CHUNK, DEG_W), jnp.float32),
        pltpu.VMEM_SHARED((DEG_PAD, DEG_W), jnp.float32),
        pltpu.SemaphoreType.DMA,
        pltpu.SemaphoreType.DMA,
        pltpu.SemaphoreType.DMA,
        pltpu.SemaphoreType.DMA,
    ],
    compiler_params=_sc_params,
)
def _deg_kernel(dst_hbm, ones_hbm, deg0_out, deg1_out, idx_v, ones_v, acc_sh,
                dsem0, dsem1, dsem2, dsem3):
    cid = lax.axis_index("c")
    sid = lax.axis_index("s")
    wid = sid * NC + cid
    pltpu.sync_copy(dst_hbm.at[wid], idx_v.at[pl.ds(0, EPW)])
    _fill_pad(idx_v, N)
    pltpu.sync_copy(ones_hbm.at[pl.ds(0, CHUNK)], ones_v)
    pltpu.sync_copy(ones_hbm.at[pl.ds(sid * RPSD, RPSD)],
                    acc_sh.at[pl.ds(sid * RPSD, RPSD)])
    plsc.subcore_barrier()

    def d_idx(g):
        return acc_sh.at[idx_v.at[pl.ds(g * CHUNK, CHUNK)]]

    dsems = [dsem0, dsem1, dsem2, dsem3]
    for b in range(4):
        pltpu.async_copy(ones_v, d_idx(b), dsems[b], add=True)

    def body(t, carry):
        for b in range(4):
            g = 4 * t + 4 + b
            pltpu.make_async_copy(ones_v, d_idx(g - 4), dsems[b]).wait()
            pltpu.async_copy(ones_v, d_idx(g), dsems[b], add=True)
        return carry

    lax.fori_loop(0, (NCHUNK - 4) // 4, body, 0)
    for b in range(4):
        g = NCHUNK - 4 + b
        pltpu.make_async_copy(ones_v, d_idx(g), dsems[b]).wait()
    plsc.subcore_barrier()

    @pl.when(cid == 0)
    def _():
        pltpu.sync_copy(acc_sh.at[pl.ds(sid * RPSD, RPSD)],
                        deg0_out.at[pl.ds(sid * RPSD, RPSD)])

    @pl.when(cid == 1)
    def _():
        pltpu.sync_copy(acc_sh.at[pl.ds(sid * RPSD, RPSD)],
                        deg1_out.at[pl.ds(sid * RPSD, RPSD)])


def _make_edge_kernel(d_feat):
    @functools.partial(
        pl.kernel,
        out_type=[jax.ShapeDtypeStruct((N, d_feat), jnp.float32),
                  jax.ShapeDtypeStruct((N, d_feat), jnp.float32)],
        mesh=_mesh,
        scratch_types=(
            [pltpu.VMEM((EPW_PAD,), jnp.int32),
             pltpu.VMEM((EPW_PAD,), jnp.int32)]
            + [pltpu.VMEM((CHUNK, d_feat), jnp.float32)] * 8
            + [pltpu.VMEM_SHARED((ACC_N, d_feat), jnp.float32)]
            + [pltpu.SemaphoreType.DMA] * 16
        ),
        compiler_params=_sc_params,
    )
    def edge_kernel(hs_hbm, src_hbm, dst_hbm, acc0_out, acc1_out,
                    src_v, dst_v, *ring):
        bufs = ring[0:8]
        acc_sh = ring[8]
        gsems = ring[9:17]
        ssems = ring[17:25]
        cid = lax.axis_index("c")
        sid = lax.axis_index("s")
        wid = sid * NC + cid
        pltpu.sync_copy(src_hbm.at[wid], src_v.at[pl.ds(0, EPW)])
        pltpu.sync_copy(dst_hbm.at[wid], dst_v.at[pl.ds(0, EPW)])
        _fill_pad(src_v, 0)
        _fill_pad(dst_v, N)
        pltpu.sync_copy(hs_hbm.at[pl.ds(sid * RPS, RPS)],
                        acc_sh.at[pl.ds(sid * RPS, RPS)])

        @pl.when(sid == NS - 1)
        def _():
            pltpu.sync_copy(hs_hbm.at[pl.ds(NS * RPS, RPS_TAIL)],
                            acc_sh.at[pl.ds(NS * RPS, RPS_TAIL)])

        plsc.subcore_barrier()

        def s_idx(g):
            return hs_hbm.at[src_v.at[pl.ds(g * CHUNK, CHUNK)]]

        def d_idx(g):
            return acc_sh.at[dst_v.at[pl.ds(g * CHUNK, CHUNK)]]

        NB = 8

        def wait_gather(g, b):
            pltpu.make_async_copy(s_idx(g), bufs[b], gsems[b]).wait()

        def wait_scatter(g, b):
            pltpu.make_async_copy(bufs[b], d_idx(g), ssems[b]).wait()

        for b in range(4):
            pltpu.async_copy(s_idx(b), bufs[b], gsems[b])
        for g in range(4):
            wait_gather(g, g)
            pltpu.async_copy(bufs[g], d_idx(g), ssems[g], add=True)
            b2 = (g + 4) % NB
            pltpu.async_copy(s_idx(g + 4), bufs[b2], gsems[b2])

        def body(t, carry):
            for b in range(NB):
                g = NB * t + 4 + b
                bb = (4 + b) % NB
                wait_gather(g, bb)
                pltpu.async_copy(bufs[bb], d_idx(g), ssems[bb], add=True)
                wait_scatter(g - 4, b)
                pltpu.async_copy(s_idx(g + 4), bufs[b], gsems[b])
            return carry

        lax.fori_loop(0, (NCHUNK - NB) // NB, body, 0)
        for g in range(NCHUNK - 4, NCHUNK):
            b = g % NB
            wait_gather(g, b)
            pltpu.async_copy(bufs[b], d_idx(g), ssems[b], add=True)
        for g in range(NCHUNK - NB, NCHUNK):
            b = g % NB
            wait_scatter(g, b)

        plsc.subcore_barrier()

        @pl.when(cid == 0)
        def _():
            pltpu.sync_copy(acc_sh.at[pl.ds(sid * RPS, RPS)],
                            acc0_out.at[pl.ds(sid * RPS, RPS)])

            @pl.when(sid == NS - 1)
            def _():
                pltpu.sync_copy(acc_sh.at[pl.ds(NS * RPS, RPS_TAIL)],
                                acc0_out.at[pl.ds(NS * RPS, RPS_TAIL)])

        @pl.when(cid == 1)
        def _():
            pltpu.sync_copy(acc_sh.at[pl.ds(sid * RPS, RPS)],
                            acc1_out.at[pl.ds(sid * RPS, RPS)])

            @pl.when(sid == NS - 1)
            def _():
                pltpu.sync_copy(acc_sh.at[pl.ds(NS * RPS, RPS_TAIL)],
                                acc1_out.at[pl.ds(NS * RPS, RPS_TAIL)])

    return edge_kernel


_edge64 = _make_edge_kernel(64)
_edge32 = _make_edge_kernel(32)


BN = 2000


def _dinv(deg0_ref, deg1_ref):
    return lax.rsqrt(deg0_ref[:, :1] + deg1_ref[:, :1] - 1.0)


def _deg_spec():
    return pl.BlockSpec((BN, DEG_W), lambda i: (i, 0))


def _tc_first(deg0, deg1, x, w0):
    def body(deg0_ref, deg1_ref, x_ref, w_ref, out_ref):
        dinv = _dinv(deg0_ref, deg1_ref)
        h = jnp.dot(x_ref[...], w_ref[...], preferred_element_type=jnp.float32)
        out_ref[...] = dinv * h

    d_out = w0.shape[1]
    return pl.pallas_call(
        body,
        grid=(N // BN,),
        in_specs=[
            _deg_spec(),
            _deg_spec(),
            pl.BlockSpec((BN, F_IN), lambda i: (i, 0)),
            pl.BlockSpec((F_IN, d_out), lambda i: (0, 0)),
        ],
        out_specs=pl.BlockSpec((BN, d_out), lambda i: (i, 0)),
        out_shape=jax.ShapeDtypeStruct((N, d_out), jnp.float32),
    )(deg0, deg1, x, w0)


def _tc_mid(deg0, deg1, acc0, acc1, hs, w, b):
    d_in = hs.shape[1]
    d_out = w.shape[1]

    def body(deg0_ref, deg1_ref, a0_ref, a1_ref, hs_ref, w_ref, b_ref,
             out_ref):
        dinv = _dinv(deg0_ref, deg1_ref)
        a = a0_ref[...] + a1_ref[...] - hs_ref[...]
        h = jnp.maximum(dinv * a + b_ref[...], 0.0)
        hn = jnp.dot(h, w_ref[...], preferred_element_type=jnp.float32)
        out_ref[...] = dinv * hn

    return pl.pallas_call(
        body,
        grid=(N // BN,),
        in_specs=[
            _deg_spec(),
            _deg_spec(),
            pl.BlockSpec((BN, d_in), lambda i: (i, 0)),
            pl.BlockSpec((BN, d_in), lambda i: (i, 0)),
            pl.BlockSpec((BN, d_in), lambda i: (i, 0)),
            pl.BlockSpec((d_in, d_out), lambda i: (0, 0)),
            pl.BlockSpec((1, d_in), lambda i: (0, 0)),
        ],
        out_specs=pl.BlockSpec((BN, d_out), lambda i: (i, 0)),
        out_shape=jax.ShapeDtypeStruct((N, d_out), jnp.float32),
    )(deg0, deg1, acc0, acc1, hs, w, b)


def _tc_last(deg0, deg1, acc0, acc1, hs, b):
    d_in = hs.shape[1]

    def body(deg0_ref, deg1_ref, a0_ref, a1_ref, hs_ref, b_ref, out_ref):
        dinv = _dinv(deg0_ref, deg1_ref)
        a = a0_ref[...] + a1_ref[...] - hs_ref[...]
        out_ref[...] = jnp.maximum(dinv * a + b_ref[...], 0.0)

    return pl.pallas_call(
        body,
        grid=(N // BN,),
        in_specs=[
            _deg_spec(),
            _deg_spec(),
            pl.BlockSpec((BN, d_in), lambda i: (i, 0)),
            pl.BlockSpec((BN, d_in), lambda i: (i, 0)),
            pl.BlockSpec((BN, d_in), lambda i: (i, 0)),
            pl.BlockSpec((1, d_in), lambda i: (0, 0)),
        ],
        out_specs=pl.BlockSpec((BN, d_in), lambda i: (i, 0)),
        out_shape=jax.ShapeDtypeStruct((N, d_in), jnp.float32),
    )(deg0, deg1, acc0, acc1, hs, b)


def kernel(x, edge_index, batch, W0, b0, W1, b1, W2, b2):
    src = edge_index[0].reshape(NW, EPW)
    dst = edge_index[1].reshape(NW, EPW)
    ones = jnp.ones((DEG_PAD, DEG_W), jnp.float32)

    deg0, deg1 = _deg_kernel(dst, ones)

    hs1 = _tc_first(deg0, deg1, x, W0)
    a0, a1 = _edge64(hs1, src, dst)
    hs2 = _tc_mid(deg0, deg1, a0, a1, hs1, W1, b0.reshape(1, -1))
    a0, a1 = _edge32(hs2, src, dst)
    hs3 = _tc_mid(deg0, deg1, a0, a1, hs2, W2, b1.reshape(1, -1))
    a0, a1 = _edge32(hs3, src, dst)
    return _tc_last(deg0, deg1, a0, a1, hs3, b2.reshape(1, -1))


# speedup vs baseline: 1.3844x; 1.0353x over previous
import functools

import jax
import jax.numpy as jnp
from jax import lax
from jax.experimental import pallas as pl
from jax.experimental.pallas import tpu as pltpu
from jax.experimental.pallas import tpu_sc as plsc

N = 10000
E = 320000
F_IN = 128

NC = 2
NS = 16
NW = NC * NS

EPW = E // NW
CHUNK = 128
NCHUNK = 80
NPAIR = NCHUNK // 2
EPW_PAD = NCHUNK * CHUNK
PADW = EPW_PAD - EPW
ACC_N = N + PADW
RPS = 624
RPS_TAIL = N - NS * RPS

DEG_PAD = 10240
DEG_W = 8
RPSD = DEG_PAD // NS

_mesh = plsc.VectorSubcoreMesh(
    core_axis_name="c", subcore_axis_name="s", num_cores=NC, num_subcores=NS)

_sc_params = pltpu.CompilerParams(use_tc_tiling_on_sc=False)


def _fill_pad(idx_v, base_val):
    for i in range(PADW // 16):
        idx_v[pl.ds(EPW + 16 * i, 16)] = (
            lax.iota(jnp.int32, 16) + (base_val + 16 * i))


@functools.partial(
    pl.kernel,
    out_type=[jax.ShapeDtypeStruct((DEG_PAD, DEG_W), jnp.float32),
              jax.ShapeDtypeStruct((DEG_PAD, DEG_W), jnp.float32)],
    mesh=_mesh,
    scratch_types=[
        pltpu.VMEM((EPW_PAD,), jnp.int32),
        pltpu.VMEM((CHUNK, DEG_W), jnp.float32),
        pltpu.VMEM_SHARED((DEG_PAD, DEG_W), jnp.float32),
        pltpu.SemaphoreType.DMA,
        pltpu.SemaphoreType.DMA,
        pltpu.SemaphoreType.DMA,
        pltpu.SemaphoreType.DMA,
    ],
    compiler_params=_sc_params,
)
def _deg_kernel(ei_hbm, ones_hbm, deg0_out, deg1_out, idx_v, ones_v, acc_sh,
                dsem0, dsem1, dsem2, dsem3):
    cid = lax.axis_index("c")
    sid = lax.axis_index("s")
    wid = sid * NC + cid
    pltpu.sync_copy(ei_hbm.at[1, pl.ds(wid * EPW, EPW)],
                    idx_v.at[pl.ds(0, EPW)])
    _fill_pad(idx_v, N)
    pltpu.sync_copy(ones_hbm.at[pl.ds(0, CHUNK)], ones_v)
    pltpu.sync_copy(ones_hbm.at[pl.ds(sid * RPSD, RPSD)],
                    acc_sh.at[pl.ds(sid * RPSD, RPSD)])
    plsc.subcore_barrier()

    def d_idx(g):
        return acc_sh.at[idx_v.at[pl.ds(g * CHUNK, CHUNK)]]

    dsems = [dsem0, dsem1, dsem2, dsem3]
    for b in range(4):
        pltpu.async_copy(ones_v, d_idx(b), dsems[b], add=True)

    def body(t, carry):
        for b in range(4):
            g = 4 * t + 4 + b
            pltpu.make_async_copy(ones_v, d_idx(g - 4), dsems[b]).wait()
            pltpu.async_copy(ones_v, d_idx(g), dsems[b], add=True)
        return carry

    lax.fori_loop(0, (NCHUNK - 4) // 4, body, 0)
    for b in range(4):
        g = NCHUNK - 4 + b
        pltpu.make_async_copy(ones_v, d_idx(g), dsems[b]).wait()
    plsc.subcore_barrier()

    @pl.when(cid == 0)
    def _():
        pltpu.sync_copy(acc_sh.at[pl.ds(sid * RPSD, RPSD)],
                        deg0_out.at[pl.ds(sid * RPSD, RPSD)])

    @pl.when(cid == 1)
    def _():
        pltpu.sync_copy(acc_sh.at[pl.ds(sid * RPSD, RPSD)],
                        deg1_out.at[pl.ds(sid * RPSD, RPSD)])


def _make_edge_kernel(d_feat):
    @functools.partial(
        pl.kernel,
        out_type=[jax.ShapeDtypeStruct((N, d_feat), jnp.float32),
                  jax.ShapeDtypeStruct((N, d_feat), jnp.float32)],
        mesh=_mesh,
        scratch_types=(
            [pltpu.VMEM((EPW_PAD,), jnp.int32),
             pltpu.VMEM((EPW_PAD,), jnp.int32)]
            + [pltpu.VMEM((CHUNK, d_feat), jnp.float32)] * 8
            + [pltpu.VMEM_SHARED((ACC_N, d_feat), jnp.float32)]
            + [pltpu.SemaphoreType.DMA] * 16
        ),
        compiler_params=_sc_params,
    )
    def edge_kernel(hs_hbm, ei_hbm, acc0_out, acc1_out,
                    src_v, dst_v, *ring):
        bufs = ring[0:8]
        acc_sh = ring[8]
        gsems = ring[9:17]
        ssems = ring[17:25]
        cid = lax.axis_index("c")
        sid = lax.axis_index("s")
        wid = sid * NC + cid
        pltpu.sync_copy(ei_hbm.at[0, pl.ds(wid * EPW, EPW)],
                        src_v.at[pl.ds(0, EPW)])
        pltpu.sync_copy(ei_hbm.at[1, pl.ds(wid * EPW, EPW)],
                        dst_v.at[pl.ds(0, EPW)])
        _fill_pad(src_v, 0)
        _fill_pad(dst_v, N)
        pltpu.sync_copy(hs_hbm.at[pl.ds(sid * RPS, RPS)],
                        acc_sh.at[pl.ds(sid * RPS, RPS)])

        @pl.when(sid == NS - 1)
        def _():
            pltpu.sync_copy(hs_hbm.at[pl.ds(NS * RPS, RPS_TAIL)],
                            acc_sh.at[pl.ds(NS * RPS, RPS_TAIL)])

        plsc.subcore_barrier()

        def s_idx(g):
            return hs_hbm.at[src_v.at[pl.ds(g * CHUNK, CHUNK)]]

        def d_idx(g):
            return acc_sh.at[dst_v.at[pl.ds(g * CHUNK, CHUNK)]]

        NB = 8

        def wait_gather(g, b):
            pltpu.make_async_copy(s_idx(g), bufs[b], gsems[b]).wait()

        def wait_scatter(g, b):
            pltpu.make_async_copy(bufs[b], d_idx(g), ssems[b]).wait()

        for b in range(4):
            pltpu.async_copy(s_idx(b), bufs[b], gsems[b])
        for g in range(4):
            wait_gather(g, g)
            pltpu.async_copy(bufs[g], d_idx(g), ssems[g], add=True)
            b2 = (g + 4) % NB
            pltpu.async_copy(s_idx(g + 4), bufs[b2], gsems[b2])

        def body(t, carry):
            for b in range(NB):
                g = NB * t + 4 + b
                bb = (4 + b) % NB
                wait_gather(g, bb)
                pltpu.async_copy(bufs[bb], d_idx(g), ssems[bb], add=True)
                wait_scatter(g - 4, b)
                pltpu.async_copy(s_idx(g + 4), bufs[b], gsems[b])
            return carry

        lax.fori_loop(0, (NCHUNK - NB) // NB, body, 0)
        for g in range(NCHUNK - 4, NCHUNK):
            b = g % NB
            wait_gather(g, b)
            pltpu.async_copy(bufs[b], d_idx(g), ssems[b], add=True)
        for g in range(NCHUNK - NB, NCHUNK):
            b = g % NB
            wait_scatter(g, b)

        plsc.subcore_barrier()

        @pl.when(cid == 0)
        def _():
            pltpu.sync_copy(acc_sh.at[pl.ds(sid * RPS, RPS)],
                            acc0_out.at[pl.ds(sid * RPS, RPS)])

            @pl.when(sid == NS - 1)
            def _():
                pltpu.sync_copy(acc_sh.at[pl.ds(NS * RPS, RPS_TAIL)],
                                acc0_out.at[pl.ds(NS * RPS, RPS_TAIL)])

        @pl.when(cid == 1)
        def _():
            pltpu.sync_copy(acc_sh.at[pl.ds(sid * RPS, RPS)],
                            acc1_out.at[pl.ds(sid * RPS, RPS)])

            @pl.when(sid == NS - 1)
            def _():
                pltpu.sync_copy(acc_sh.at[pl.ds(NS * RPS, RPS_TAIL)],
                                acc1_out.at[pl.ds(NS * RPS, RPS_TAIL)])

    return edge_kernel


_edge64 = _make_edge_kernel(64)
_edge32 = _make_edge_kernel(32)


BN = 2000


def _dinv(deg0_ref, deg1_ref):
    return lax.rsqrt(deg0_ref[:, :1] + deg1_ref[:, :1] - 1.0)


def _deg_spec():
    return pl.BlockSpec((BN, DEG_W), lambda i: (i, 0))


def _tc_matmul(x, w0):
    def body(x_ref, w_ref, out_ref):
        out_ref[...] = jnp.dot(x_ref[...], w_ref[...],
                               preferred_element_type=jnp.float32)

    d_out = w0.shape[1]
    return pl.pallas_call(
        body,
        grid=(N // BN,),
        in_specs=[
            pl.BlockSpec((BN, F_IN), lambda i: (i, 0)),
            pl.BlockSpec((F_IN, d_out), lambda i: (0, 0)),
        ],
        out_specs=pl.BlockSpec((BN, d_out), lambda i: (i, 0)),
        out_shape=jax.ShapeDtypeStruct((N, d_out), jnp.float32),
    )(x, w0)


def _tc_scale(deg0, deg1, h):
    d_out = h.shape[1]

    def body(deg0_ref, deg1_ref, h_ref, out_ref):
        out_ref[...] = _dinv(deg0_ref, deg1_ref) * h_ref[...]

    return pl.pallas_call(
        body,
        grid=(N // BN,),
        in_specs=[
            _deg_spec(),
            _deg_spec(),
            pl.BlockSpec((BN, d_out), lambda i: (i, 0)),
        ],
        out_specs=pl.BlockSpec((BN, d_out), lambda i: (i, 0)),
        out_shape=jax.ShapeDtypeStruct((N, d_out), jnp.float32),
    )(deg0, deg1, h)


def _tc_mid(deg0, deg1, acc0, acc1, hs, w, b):
    d_in = hs.shape[1]
    d_out = w.shape[1]

    def body(deg0_ref, deg1_ref, a0_ref, a1_ref, hs_ref, w_ref, b_ref,
             out_ref):
        dinv = _dinv(deg0_ref, deg1_ref)
        a = a0_ref[...] + a1_ref[...] - hs_ref[...]
        h = jnp.maximum(dinv * a + b_ref[...], 0.0)
        hn = jnp.dot(h, w_ref[...], preferred_element_type=jnp.float32)
        out_ref[...] = dinv * hn

    return pl.pallas_call(
        body,
        grid=(N // BN,),
        in_specs=[
            _deg_spec(),
            _deg_spec(),
            pl.BlockSpec((BN, d_in), lambda i: (i, 0)),
            pl.BlockSpec((BN, d_in), lambda i: (i, 0)),
            pl.BlockSpec((BN, d_in), lambda i: (i, 0)),
            pl.BlockSpec((d_in, d_out), lambda i: (0, 0)),
            pl.BlockSpec((1, d_in), lambda i: (0, 0)),
        ],
        out_specs=pl.BlockSpec((BN, d_out), lambda i: (i, 0)),
        out_shape=jax.ShapeDtypeStruct((N, d_out), jnp.float32),
    )(deg0, deg1, acc0, acc1, hs, w, b)


def _tc_last(deg0, deg1, acc0, acc1, hs, b):
    d_in = hs.shape[1]

    def body(deg0_ref, deg1_ref, a0_ref, a1_ref, hs_ref, b_ref, out_ref):
        dinv = _dinv(deg0_ref, deg1_ref)
        a = a0_ref[...] + a1_ref[...] - hs_ref[...]
        out_ref[...] = jnp.maximum(dinv * a + b_ref[...], 0.0)

    return pl.pallas_call(
        body,
        grid=(N // BN,),
        in_specs=[
            _deg_spec(),
            _deg_spec(),
            pl.BlockSpec((BN, d_in), lambda i: (i, 0)),
            pl.BlockSpec((BN, d_in), lambda i: (i, 0)),
            pl.BlockSpec((BN, d_in), lambda i: (i, 0)),
            pl.BlockSpec((1, d_in), lambda i: (0, 0)),
        ],
        out_specs=pl.BlockSpec((BN, d_in), lambda i: (i, 0)),
        out_shape=jax.ShapeDtypeStruct((N, d_in), jnp.float32),
    )(deg0, deg1, acc0, acc1, hs, b)


def kernel(x, edge_index, batch, W0, b0, W1, b1, W2, b2):
    ones = jnp.ones((DEG_PAD, DEG_W), jnp.float32)

    deg0, deg1 = _deg_kernel(edge_index, ones)

    hs1 = _tc_scale(deg0, deg1, _tc_matmul(x, W0))
    a0, a1 = _edge64(hs1, edge_index)
    hs2 = _tc_mid(deg0, deg1, a0, a1, hs1, W1, b0.reshape(1, -1))
    a0, a1 = _edge32(hs2, edge_index)
    hs3 = _tc_mid(deg0, deg1, a0, a1, hs2, W2, b1.reshape(1, -1))
    a0, a1 = _edge32(hs3, edge_index)
    return _tc_last(deg0, deg1, a0, a1, hs3, b2.reshape(1, -1))
